# baseline (speedup 1.0000x reference)
import functools

import jax
import jax.numpy as jnp
from jax import lax
from jax.experimental import pallas as pl
from jax.experimental.pallas import tpu as pltpu
from jax.experimental.pallas import tpu_sc as plsc

N = 10000
E = 320000
D = 128
C = 40
B = 64

NPAD = 10240
CHUNK = 128
NW = 32
EPAD = 327680
NCH = EPAD // CHUNK
CH_PER_TILE = NCH // NW
ROWS_PER_TILE = NPAD // 16
RB = 10
RBLK = N // RB


def _zero_f32_block(ref, rows, cols):
  groups = cols // 16

  def body(t, carry):
    i = t // groups
    g = t % groups
    ref[i, pl.ds(g * 16, 16)] = jnp.zeros((16,), jnp.float32)
    return carry

  lax.fori_loop(0, rows * groups, body, None)


def _fill_f32_1d(ref, n, value):
  def body(t, carry):
    ref[pl.ds(t * 16, 16)] = jnp.full((16,), value, jnp.float32)
    return carry

  lax.fori_loop(0, n // 16, body, None)


def _sc_degree(dstc):
  mesh = plsc.VectorSubcoreMesh(core_axis_name="c", subcore_axis_name="s")

  @functools.partial(
      pl.kernel,
      out_type=jax.ShapeDtypeStruct((2 * NPAD,), jnp.float32),
      mesh=mesh,
      scratch_types=[
          pltpu.VMEM((CH_PER_TILE, CHUNK), jnp.int32),
          pltpu.VMEM((CHUNK,), jnp.float32),
          pltpu.VMEM((ROWS_PER_TILE,), jnp.float32),
          pltpu.VMEM_SHARED((NPAD,), jnp.float32),
      ],
  )
  def k(dstc_hbm, out_hbm, idxd_v, ones_v, zeros_v, deg_sh):
    c = lax.axis_index("c")
    s = lax.axis_index("s")
    wid = s * 2 + c
    _fill_f32_1d(ones_v, CHUNK, 1.0)
    _fill_f32_1d(zeros_v, ROWS_PER_TILE, 0.0)
    pltpu.sync_copy(zeros_v, deg_sh.at[pl.ds(s * ROWS_PER_TILE, ROWS_PER_TILE)])
    plsc.subcore_barrier()
    pltpu.sync_copy(dstc_hbm.at[pl.ds(wid * CH_PER_TILE, CH_PER_TILE)], idxd_v)

    def body(j, carry):
      pltpu.sync_copy(ones_v, deg_sh.at[idxd_v.at[j]], add=True)
      return carry

    lax.fori_loop(0, CH_PER_TILE, body, None)
    plsc.subcore_barrier()
    pltpu.sync_copy(
        deg_sh.at[pl.ds(s * ROWS_PER_TILE, ROWS_PER_TILE)],
        out_hbm.at[pl.ds(c * NPAD + s * ROWS_PER_TILE, ROWS_PER_TILE)],
    )

  return k(dstc)


def _sc_scatter(gs, srcc, dstc):
  mesh = plsc.VectorSubcoreMesh(core_axis_name="c", subcore_axis_name="s")

  @functools.partial(
      pl.kernel,
      out_type=jax.ShapeDtypeStruct((2 * NPAD, D), jnp.float32),
      mesh=mesh,
      scratch_types=[
          pltpu.VMEM((CH_PER_TILE // 2, CHUNK), jnp.int32),
          pltpu.VMEM((CH_PER_TILE // 2, CHUNK), jnp.int32),
          [pltpu.VMEM((CHUNK, D), jnp.float32)] * 2,
          pltpu.VMEM_SHARED((NPAD, D), jnp.float32),
          [pltpu.SemaphoreType.DMA] * 4,
      ],
  )
  def k(gs_hbm, srcc_hbm, dstc_hbm, out_hbm, idxs_v, idxd_v, bufs, s_sh, sems):
    c = lax.axis_index("c")
    s = lax.axis_index("s")
    wid = s * 2 + c
    _zero_f32_block(bufs[0], CHUNK, D)
    zdescs = [
        pltpu.async_copy(
            bufs[0],
            s_sh.at[pl.ds(s * ROWS_PER_TILE + bb * CHUNK, CHUNK)],
            sems[0])
        for bb in range(ROWS_PER_TILE // CHUNK)
    ]
    half = CH_PER_TILE // 2
    base0 = wid * CH_PER_TILE
    pltpu.sync_copy(srcc_hbm.at[pl.ds(base0, half)], idxs_v)
    pltpu.sync_copy(dstc_hbm.at[pl.ds(base0, half)], idxd_v)
    for dsc in zdescs:
      dsc.wait()
    plsc.subcore_barrier()

    for p in range(2):
      if p == 1:
        base = wid * CH_PER_TILE + half
        pltpu.sync_copy(srcc_hbm.at[pl.ds(base, half)], idxs_v)
        pltpu.sync_copy(dstc_hbm.at[pl.ds(base, half)], idxd_v)

      @pl.loop(0, half, step=2)
      def _pipe(j):
        descs = [
            pltpu.async_copy(gs_hbm.at[idxs_v.at[j + t]], bufs[t], sems[t])
            for t in range(2)
        ]
        sdescs = []
        for t in range(2):
          descs[t].wait()
          sdescs.append(pltpu.async_copy(
              bufs[t], s_sh.at[idxd_v.at[j + t]], sems[2 + t], add=True))
        for sd in sdescs:
          sd.wait()

    plsc.subcore_barrier()
    pltpu.sync_copy(
        s_sh.at[pl.ds(s * ROWS_PER_TILE, ROWS_PER_TILE)],
        out_hbm.at[pl.ds(c * NPAD + s * ROWS_PER_TILE, ROWS_PER_TILE)],
    )

  return k(gs, srcc, dstc)


def _tc_first(x, W, d0col, d1col):
  def body(x_ref, w_ref, d0_ref, d1_ref, out_ref, dis_ref):
    dis = lax.rsqrt(d0_ref[...] + d1_ref[...] + 1.0)
    dis_ref[...] = dis
    g = jnp.dot(x_ref[...], w_ref[...], preferred_element_type=jnp.float32)
    out_ref[...] = dis * g

  return pl.pallas_call(
      body,
      grid=(RB,),
      in_specs=[
          pl.BlockSpec((RBLK, D), lambda r: (r, 0)),
          pl.BlockSpec((D, D), lambda r: (0, 0)),
          pl.BlockSpec((RBLK, 1), lambda r: (r, 0)),
          pl.BlockSpec((RBLK, 1), lambda r: (r, 0)),
      ],
      out_specs=[
          pl.BlockSpec((RBLK, D), lambda r: (r, 0)),
          pl.BlockSpec((RBLK, 1), lambda r: (r, 0)),
      ],
      out_shape=[
          jax.ShapeDtypeStruct((N, D), jnp.float32),
          jax.ShapeDtypeStruct((N, 1), jnp.float32),
      ],
  )(x, W, d0col, d1col)


def _tc_mid(s0, s1, gsp, dis_col, bias, W):
  def body(s0_ref, s1_ref, gsp_ref, dis_ref, b_ref, w_ref, out_ref):
    agg = dis_ref[...] * (s0_ref[...] + s1_ref[...] + gsp_ref[...])
    h = jnp.maximum(agg + b_ref[...], 0.0)
    g = jnp.dot(h, w_ref[...], preferred_element_type=jnp.float32)
    out_ref[...] = dis_ref[...] * g

  return pl.pallas_call(
      body,
      grid=(RB,),
      in_specs=[
          pl.BlockSpec((RBLK, D), lambda r: (r, 0)),
          pl.BlockSpec((RBLK, D), lambda r: (r, 0)),
          pl.BlockSpec((RBLK, D), lambda r: (r, 0)),
          pl.BlockSpec((RBLK, 1), lambda r: (r, 0)),
          pl.BlockSpec((1, D), lambda r: (0, 0)),
          pl.BlockSpec((D, D), lambda r: (0, 0)),
      ],
      out_specs=pl.BlockSpec((RBLK, D), lambda r: (r, 0)),
      out_shape=jax.ShapeDtypeStruct((N, D), jnp.float32),
  )(s0, s1, gsp, dis_col, bias, W)


def _tc_final(s0, s1, gsp, dis_col, bias, batch3, Wl, bl):
  def body(s0_ref, s1_ref, gsp_ref, dis_ref, b_ref, bat_ref, wl_ref, bl_ref,
           out_ref, acc_ref, cnt_ref):
    r = pl.program_id(0)

    @pl.when(r == 0)
    def _():
      acc_ref[...] = jnp.zeros((B, D), jnp.float32)
      cnt_ref[...] = jnp.zeros((B, 128), jnp.float32)

    agg = dis_ref[...] * (s0_ref[...] + s1_ref[...] + gsp_ref[...])
    h = agg + b_ref[...]
    bat = bat_ref[0]
    gid = lax.broadcasted_iota(jnp.int32, (B, RBLK), 0)
    p = jnp.where(bat == gid, 1.0, 0.0)
    acc_ref[...] += jnp.dot(p, h, preferred_element_type=jnp.float32)
    cnt_ref[...] += jnp.broadcast_to(
        jnp.sum(p, axis=1, keepdims=True), (B, 128))

    @pl.when(r == RB - 1)
    def _():
      x_g = acc_ref[...] / jnp.maximum(cnt_ref[...], 1.0)
      logits = (
          jnp.dot(x_g, wl_ref[...], preferred_element_type=jnp.float32)
          + bl_ref[...])
      m = jnp.max(logits, axis=-1, keepdims=True)
      z = logits - m
      lse = jnp.log(jnp.sum(jnp.exp(z), axis=-1, keepdims=True))
      out_ref[...] = z - lse

  return pl.pallas_call(
      body,
      grid=(RB,),
      in_specs=[
          pl.BlockSpec((RBLK, D), lambda r: (r, 0)),
          pl.BlockSpec((RBLK, D), lambda r: (r, 0)),
          pl.BlockSpec((RBLK, D), lambda r: (r, 0)),
          pl.BlockSpec((RBLK, 1), lambda r: (r, 0)),
          pl.BlockSpec((1, D), lambda r: (0, 0)),
          pl.BlockSpec((1, 1, RBLK), lambda r: (r, 0, 0)),
          pl.BlockSpec((D, C), lambda r: (0, 0)),
          pl.BlockSpec((1, C), lambda r: (0, 0)),
      ],
      out_specs=pl.BlockSpec((B, C), lambda r: (0, 0)),
      out_shape=jax.ShapeDtypeStruct((B, C), jnp.float32),
      scratch_shapes=[
          pltpu.VMEM((B, D), jnp.float32),
          pltpu.VMEM((B, 128), jnp.float32),
      ],
  )(s0, s1, gsp, dis_col, bias, batch3, Wl, bl)


def kernel(x, edge_index, batch, W1, b1, W2, b2, W3, b3, Wl, bl):
  src = edge_index[0]
  dst = edge_index[1]
  pad = EPAD - E
  ar = jnp.arange(pad, dtype=jnp.int32)
  srcc = jnp.concatenate([src, ar % N]).reshape(NCH, CHUNK)
  dstc = jnp.concatenate([dst, N + (ar % (NPAD - N))]).reshape(NCH, CHUNK)

  degp = _sc_degree(dstc)
  d0col = degp[:N].reshape(N, 1)
  d1col = degp[NPAD:NPAD + N].reshape(N, 1)

  b1r = b1.reshape(1, D)
  b2r = b2.reshape(1, D)
  b3r = b3.reshape(1, D)
  blr = bl.reshape(1, C)
  batch3 = batch.reshape(RB, 1, RBLK)

  gs1, dis_col = _tc_first(x, W1, d0col, d1col)
  s = _sc_scatter(gs1, srcc, dstc)
  gs2 = _tc_mid(s[:N], s[NPAD:NPAD + N], gs1, dis_col, b1r, W2)
  s = _sc_scatter(gs2, srcc, dstc)
  gs3 = _tc_mid(s[:N], s[NPAD:NPAD + N], gs2, dis_col, b2r, W3)
  s = _sc_scatter(gs3, srcc, dstc)
  return _tc_final(s[:N], s[NPAD:NPAD + N], gs3, dis_col, b3r, batch3, Wl, blr)


# speedup vs baseline: 1.0036x; 1.0036x over previous
import functools

import jax
import jax.numpy as jnp
from jax import lax
from jax.experimental import pallas as pl
from jax.experimental.pallas import tpu as pltpu
from jax.experimental.pallas import tpu_sc as plsc

N = 10000
E = 320000
D = 128
C = 40
B = 64

NPAD = 10240
CHUNK = 128
NW = 32
EPAD = 327680
NCH = EPAD // CHUNK
CH_PER_TILE = NCH // NW
ROWS_PER_TILE = NPAD // 16
RB = 10
RBLK = N // RB


def _zero_f32_block(ref, rows, cols):
  groups = cols // 16

  def body(t, carry):
    i = t // groups
    g = t % groups
    ref[i, pl.ds(g * 16, 16)] = jnp.zeros((16,), jnp.float32)
    return carry

  lax.fori_loop(0, rows * groups, body, None)


def _fill_f32_1d(ref, n, value):
  def body(t, carry):
    ref[pl.ds(t * 16, 16)] = jnp.full((16,), value, jnp.float32)
    return carry

  lax.fori_loop(0, n // 16, body, None)


def _sc_degree(dstc):
  mesh = plsc.VectorSubcoreMesh(core_axis_name="c", subcore_axis_name="s")

  @functools.partial(
      pl.kernel,
      out_type=jax.ShapeDtypeStruct((2 * NPAD,), jnp.float32),
      mesh=mesh,
      scratch_types=[
          pltpu.VMEM((CH_PER_TILE, CHUNK), jnp.int32),
          pltpu.VMEM((CHUNK,), jnp.float32),
          pltpu.VMEM((ROWS_PER_TILE,), jnp.float32),
          pltpu.VMEM_SHARED((NPAD,), jnp.float32),
      ],
  )
  def k(dstc_hbm, out_hbm, idxd_v, ones_v, zeros_v, deg_sh):
    c = lax.axis_index("c")
    s = lax.axis_index("s")
    wid = s * 2 + c
    _fill_f32_1d(ones_v, CHUNK, 1.0)
    _fill_f32_1d(zeros_v, ROWS_PER_TILE, 0.0)
    pltpu.sync_copy(zeros_v, deg_sh.at[pl.ds(s * ROWS_PER_TILE, ROWS_PER_TILE)])
    plsc.subcore_barrier()
    pltpu.sync_copy(dstc_hbm.at[pl.ds(wid * CH_PER_TILE, CH_PER_TILE)], idxd_v)

    def body(j, carry):
      pltpu.sync_copy(ones_v, deg_sh.at[idxd_v.at[j]], add=True)
      return carry

    lax.fori_loop(0, CH_PER_TILE, body, None)
    plsc.subcore_barrier()
    pltpu.sync_copy(
        deg_sh.at[pl.ds(s * ROWS_PER_TILE, ROWS_PER_TILE)],
        out_hbm.at[pl.ds(c * NPAD + s * ROWS_PER_TILE, ROWS_PER_TILE)],
    )

  return k(dstc)


def _sc_scatter(gs, srcc, dstc):
  mesh = plsc.VectorSubcoreMesh(core_axis_name="c", subcore_axis_name="s")

  @functools.partial(
      pl.kernel,
      out_type=jax.ShapeDtypeStruct((2 * NPAD, D), jnp.float32),
      mesh=mesh,
      scratch_types=[
          pltpu.VMEM((CH_PER_TILE // 2, CHUNK), jnp.int32),
          pltpu.VMEM((CH_PER_TILE // 2, CHUNK), jnp.int32),
          [pltpu.VMEM((CHUNK, D), jnp.float32)] * 2,
          pltpu.VMEM_SHARED((NPAD, D), jnp.float32),
          [pltpu.SemaphoreType.DMA] * 4,
      ],
  )
  def k(gs_hbm, srcc_hbm, dstc_hbm, out_hbm, idxs_v, idxd_v, bufs, s_sh, sems):
    c = lax.axis_index("c")
    s = lax.axis_index("s")
    wid = s * 2 + c
    _zero_f32_block(bufs[0], CHUNK, D)
    zdescs = [
        pltpu.async_copy(
            bufs[0],
            s_sh.at[pl.ds(s * ROWS_PER_TILE + bb * CHUNK, CHUNK)],
            sems[0])
        for bb in range(ROWS_PER_TILE // CHUNK)
    ]
    half = CH_PER_TILE // 2
    base0 = wid * CH_PER_TILE
    pltpu.sync_copy(srcc_hbm.at[pl.ds(base0, half)], idxs_v)
    pltpu.sync_copy(dstc_hbm.at[pl.ds(base0, half)], idxd_v)
    for dsc in zdescs:
      dsc.wait()
    plsc.subcore_barrier()

    for p in range(2):
      if p == 1:
        base = wid * CH_PER_TILE + half
        pltpu.sync_copy(srcc_hbm.at[pl.ds(base, half)], idxs_v)
        pltpu.sync_copy(dstc_hbm.at[pl.ds(base, half)], idxd_v)

      @pl.loop(0, half, step=2)
      def _pipe(j):
        descs = [
            pltpu.async_copy(gs_hbm.at[idxs_v.at[j + t]], bufs[t], sems[t])
            for t in range(2)
        ]
        sdescs = []
        for t in range(2):
          descs[t].wait()
          sdescs.append(pltpu.async_copy(
              bufs[t], s_sh.at[idxd_v.at[j + t]], sems[2 + t], add=True))
        for sd in sdescs:
          sd.wait()

    plsc.subcore_barrier()
    pltpu.sync_copy(
        s_sh.at[pl.ds(s * ROWS_PER_TILE, ROWS_PER_TILE)],
        out_hbm.at[pl.ds(c * NPAD + s * ROWS_PER_TILE, ROWS_PER_TILE)],
    )

  return k(gs, srcc, dstc)


def _tc_dis(d0, d1):
  def body(d0_ref, d1_ref, out_ref):
    out_ref[...] = lax.rsqrt(d0_ref[...] + d1_ref[...] + 1.0)

  return pl.pallas_call(
      body,
      out_shape=jax.ShapeDtypeStruct((NPAD // 128, 128), jnp.float32),
  )(d0, d1)


def _tc_first(x, W, dis_col):
  def body(x_ref, w_ref, dis_ref, out_ref):
    g = jnp.dot(x_ref[...], w_ref[...], preferred_element_type=jnp.float32)
    out_ref[...] = dis_ref[...] * g

  return pl.pallas_call(
      body,
      grid=(RB,),
      in_specs=[
          pl.BlockSpec((RBLK, D), lambda r: (r, 0)),
          pl.BlockSpec((D, D), lambda r: (0, 0)),
          pl.BlockSpec((RBLK, 1), lambda r: (r, 0)),
      ],
      out_specs=pl.BlockSpec((RBLK, D), lambda r: (r, 0)),
      out_shape=jax.ShapeDtypeStruct((N, D), jnp.float32),
  )(x, W, dis_col)


def _tc_mid(s0, s1, gsp, dis_col, bias, W):
  def body(s0_ref, s1_ref, gsp_ref, dis_ref, b_ref, w_ref, out_ref):
    agg = dis_ref[...] * (s0_ref[...] + s1_ref[...] + gsp_ref[...])
    h = jnp.maximum(agg + b_ref[...], 0.0)
    g = jnp.dot(h, w_ref[...], preferred_element_type=jnp.float32)
    out_ref[...] = dis_ref[...] * g

  return pl.pallas_call(
      body,
      grid=(RB,),
      in_specs=[
          pl.BlockSpec((RBLK, D), lambda r: (r, 0)),
          pl.BlockSpec((RBLK, D), lambda r: (r, 0)),
          pl.BlockSpec((RBLK, D), lambda r: (r, 0)),
          pl.BlockSpec((RBLK, 1), lambda r: (r, 0)),
          pl.BlockSpec((1, D), lambda r: (0, 0)),
          pl.BlockSpec((D, D), lambda r: (0, 0)),
      ],
      out_specs=pl.BlockSpec((RBLK, D), lambda r: (r, 0)),
      out_shape=jax.ShapeDtypeStruct((N, D), jnp.float32),
  )(s0, s1, gsp, dis_col, bias, W)


def _tc_final(s0, s1, gsp, dis_col, bias, batch3, Wl, bl):
  def body(s0_ref, s1_ref, gsp_ref, dis_ref, b_ref, bat_ref, wl_ref, bl_ref,
           out_ref, acc_ref, cnt_ref):
    r = pl.program_id(0)

    @pl.when(r == 0)
    def _():
      acc_ref[...] = jnp.zeros((B, D), jnp.float32)
      cnt_ref[...] = jnp.zeros((B, 128), jnp.float32)

    agg = dis_ref[...] * (s0_ref[...] + s1_ref[...] + gsp_ref[...])
    h = agg + b_ref[...]
    bat = bat_ref[0]
    gid = lax.broadcasted_iota(jnp.int32, (B, RBLK), 0)
    p = jnp.where(bat == gid, 1.0, 0.0)
    acc_ref[...] += jnp.dot(p, h, preferred_element_type=jnp.float32)
    cnt_ref[...] += jnp.broadcast_to(
        jnp.sum(p, axis=1, keepdims=True), (B, 128))

    @pl.when(r == RB - 1)
    def _():
      x_g = acc_ref[...] / jnp.maximum(cnt_ref[...], 1.0)
      logits = (
          jnp.dot(x_g, wl_ref[...], preferred_element_type=jnp.float32)
          + bl_ref[...])
      m = jnp.max(logits, axis=-1, keepdims=True)
      z = logits - m
      lse = jnp.log(jnp.sum(jnp.exp(z), axis=-1, keepdims=True))
      out_ref[...] = z - lse

  return pl.pallas_call(
      body,
      grid=(RB,),
      in_specs=[
          pl.BlockSpec((RBLK, D), lambda r: (r, 0)),
          pl.BlockSpec((RBLK, D), lambda r: (r, 0)),
          pl.BlockSpec((RBLK, D), lambda r: (r, 0)),
          pl.BlockSpec((RBLK, 1), lambda r: (r, 0)),
          pl.BlockSpec((1, D), lambda r: (0, 0)),
          pl.BlockSpec((1, 1, RBLK), lambda r: (r, 0, 0)),
          pl.BlockSpec((D, C), lambda r: (0, 0)),
          pl.BlockSpec((1, C), lambda r: (0, 0)),
      ],
      out_specs=pl.BlockSpec((B, C), lambda r: (0, 0)),
      out_shape=jax.ShapeDtypeStruct((B, C), jnp.float32),
      scratch_shapes=[
          pltpu.VMEM((B, D), jnp.float32),
          pltpu.VMEM((B, 128), jnp.float32),
      ],
  )(s0, s1, gsp, dis_col, bias, batch3, Wl, bl)


def kernel(x, edge_index, batch, W1, b1, W2, b2, W3, b3, Wl, bl):
  src = edge_index[0]
  dst = edge_index[1]
  pad = EPAD - E
  ar = jnp.arange(pad, dtype=jnp.int32)
  srcc = jnp.concatenate([src, ar % N]).reshape(NCH, CHUNK)
  dstc = jnp.concatenate([dst, N + (ar % (NPAD - N))]).reshape(NCH, CHUNK)

  degp = _sc_degree(dstc)
  d0 = degp[:NPAD].reshape(NPAD // 128, 128)
  d1 = degp[NPAD:].reshape(NPAD // 128, 128)
  dis_col = _tc_dis(d0, d1).reshape(NPAD)[:N].reshape(N, 1)

  b1r = b1.reshape(1, D)
  b2r = b2.reshape(1, D)
  b3r = b3.reshape(1, D)
  blr = bl.reshape(1, C)
  batch3 = batch.reshape(RB, 1, RBLK)

  gs1 = _tc_first(x, W1, dis_col)
  s = _sc_scatter(gs1, srcc, dstc)
  gs2 = _tc_mid(s[:N], s[NPAD:NPAD + N], gs1, dis_col, b1r, W2)
  s = _sc_scatter(gs2, srcc, dstc)
  gs3 = _tc_mid(s[:N], s[NPAD:NPAD + N], gs2, dis_col, b2r, W3)
  s = _sc_scatter(gs3, srcc, dstc)
  return _tc_final(s[:N], s[NPAD:NPAD + N], gs3, dis_col, b3r, batch3, Wl, blr)
